# Initial kernel scaffold; baseline (speedup 1.0000x reference)
import jax
import jax.numpy as jnp
from jax.experimental import pallas as pl


def kernel(x, edge_index, W_lin0, b_lin0, W_self0, b_self0, W_lin1, b_lin1, W_self1, b_self1, W_lin2, b_lin2, W_self2, b_self2, W_mp1, b_mp1, W_mp2, b_mp2):
    raise NotImplementedError("write your pallas kernel here")


# speedup vs baseline: 3.5729x; 3.5729x over previous
import functools

import jax
import jax.numpy as jnp
from jax import lax
from jax.experimental import pallas as pl
from jax.experimental.pallas import tpu as pltpu
from jax.experimental.pallas import tpu_sc as plsc

_N = 10000
_D = 128
_E = 320000

_NC = 2
_NS = 16
_NW = _NC * _NS
_CHUNK = 128
_CHUNKS_PER_TILE = 79
_PER_TILE = _CHUNK * _CHUNKS_PER_TILE
_E_PAD = _PER_TILE * _NW
_ACC_ROWS = 10240
_ZERO_ROWS_PER_TILE = _ACC_ROWS // _NS
_OUT_ROWS_PER_TILE = _ACC_ROWS // _NS

_BN = 1000


def _lin_pair_first_body(h_ref, wl_ref, bl_ref, ws_ref, bs_ref, m_ref, s_ref):
    h = h_ref[...]
    m_ref[...] = jnp.dot(h, wl_ref[...], preferred_element_type=jnp.float32) + bl_ref[...]
    s_ref[...] = jnp.dot(h, ws_ref[...], preferred_element_type=jnp.float32) + bs_ref[...]


def _lin_pair_next_body(sp_ref, agg_ref, wl_ref, bl_ref, ws_ref, bs_ref, m_ref, s_ref):
    h = jnp.maximum(sp_ref[...] + agg_ref[0] + agg_ref[1], 0.0)
    m_ref[...] = jnp.dot(h, wl_ref[...], preferred_element_type=jnp.float32) + bl_ref[...]
    s_ref[...] = jnp.dot(h, ws_ref[...], preferred_element_type=jnp.float32) + bs_ref[...]


def _head_body(sp_ref, agg_ref, w1_ref, b1_ref, w2_ref, b2_ref, emb_ref, out_ref):
    emb = sp_ref[...] + agg_ref[0] + agg_ref[1]
    emb_ref[...] = emb
    h = jnp.maximum(emb, 0.0)
    t = jnp.maximum(
        jnp.dot(h, w1_ref[...], preferred_element_type=jnp.float32) + b1_ref[...], 0.0)
    out_ref[...] = jnp.dot(t, w2_ref[...], preferred_element_type=jnp.float32) + b2_ref[...]


_ROW_SPEC = pl.BlockSpec((_BN, _D), lambda i: (i, 0))
_W_SPEC = pl.BlockSpec((_D, _D), lambda i: (0, 0))
_B_SPEC = pl.BlockSpec((1, _D), lambda i: (0, 0))
_AGG_SPEC = pl.BlockSpec((_NC, _BN, _D), lambda i: (0, i, 0))
_GRID = (_N // _BN,)
_ND_OUT = jax.ShapeDtypeStruct((_N, _D), jnp.float32)


def _lin_pair_first(h, wl, bl, ws, bs):
    return pl.pallas_call(
        _lin_pair_first_body,
        grid=_GRID,
        in_specs=[_ROW_SPEC, _W_SPEC, _B_SPEC, _W_SPEC, _B_SPEC],
        out_specs=[_ROW_SPEC, _ROW_SPEC],
        out_shape=[_ND_OUT, _ND_OUT],
    )(h, wl, bl, ws, bs)


def _lin_pair_next(s_prev, agg, wl, bl, ws, bs):
    return pl.pallas_call(
        _lin_pair_next_body,
        grid=_GRID,
        in_specs=[_ROW_SPEC, _AGG_SPEC, _W_SPEC, _B_SPEC, _W_SPEC, _B_SPEC],
        out_specs=[_ROW_SPEC, _ROW_SPEC],
        out_shape=[_ND_OUT, _ND_OUT],
    )(s_prev, agg, wl, bl, ws, bs)


def _head(s_prev, agg, w1, b1, w2, b2):
    return pl.pallas_call(
        _head_body,
        grid=_GRID,
        in_specs=[_ROW_SPEC, _AGG_SPEC, _W_SPEC, _B_SPEC, _W_SPEC, _B_SPEC],
        out_specs=[_ROW_SPEC, _ROW_SPEC],
        out_shape=[_ND_OUT, _ND_OUT],
    )(s_prev, agg, w1, b1, w2, b2)


def _segsum_sc_body(m_hbm, src_hbm, dst_hbm, out_hbm,
                    sidx_v, didx_v, rows_v, zeros_v, acc, sem):
    c = lax.axis_index("c")
    s = lax.axis_index("s")
    wid = s * _NC + c

    zv = jnp.zeros((16,), jnp.float32)

    def zrow(r, carry):
        for cc in range(_D // 16):
            zeros_v[r, pl.ds(cc * 16, 16)] = zv
        return carry

    lax.fori_loop(0, _CHUNK, zrow, 0)
    zbase = s * _ZERO_ROWS_PER_TILE
    for z in range(_ZERO_ROWS_PER_TILE // _CHUNK):
        pltpu.sync_copy(zeros_v, acc.at[pl.ds(zbase + z * _CHUNK, _CHUNK)])
    plsc.subcore_barrier()

    ebase = wid * _PER_TILE

    def body(j, carry):
        off = ebase + j * _CHUNK
        pltpu.sync_copy(src_hbm.at[pl.ds(off, _CHUNK)], sidx_v)
        pltpu.sync_copy(dst_hbm.at[pl.ds(off, _CHUNK)], didx_v)
        pltpu.async_copy(m_hbm.at[sidx_v], rows_v, sem).wait()
        pltpu.sync_copy(rows_v, acc.at[didx_v], add=True)
        return carry

    lax.fori_loop(0, _CHUNKS_PER_TILE, body, 0)
    plsc.subcore_barrier()

    obase = s * _OUT_ROWS_PER_TILE
    pltpu.sync_copy(acc.at[pl.ds(obase, _OUT_ROWS_PER_TILE)],
                    out_hbm.at[c, pl.ds(obase, _OUT_ROWS_PER_TILE)])


def _segsum(m, src_p, dst_p):
    mesh = plsc.VectorSubcoreMesh(core_axis_name="c", subcore_axis_name="s")
    fn = functools.partial(
        pl.kernel,
        mesh=mesh,
        out_type=jax.ShapeDtypeStruct((_NC, _ACC_ROWS, _D), jnp.float32),
        scratch_types=[
            pltpu.VMEM((_CHUNK,), jnp.int32),
            pltpu.VMEM((_CHUNK,), jnp.int32),
            pltpu.VMEM((_CHUNK, _D), jnp.float32),
            pltpu.VMEM((_CHUNK, _D), jnp.float32),
            pltpu.VMEM_SHARED((_ACC_ROWS, _D), jnp.float32),
            pltpu.SemaphoreType.DMA,
        ],
    )(_segsum_sc_body)
    return fn(m, src_p, dst_p)


def kernel(x, edge_index,
           W_lin0, b_lin0, W_self0, b_self0,
           W_lin1, b_lin1, W_self1, b_self1,
           W_lin2, b_lin2, W_self2, b_self2,
           W_mp1, b_mp1, W_mp2, b_mp2):
    src = edge_index[0]
    dst = edge_index[1]
    pad = _E_PAD - _E
    src_p = jnp.concatenate([src, jnp.zeros((pad,), jnp.int32)])
    dst_p = jnp.concatenate([dst, jnp.full((pad,), _N, jnp.int32)])

    bl0 = b_lin0.reshape(1, _D)
    bs0 = b_self0.reshape(1, _D)
    bl1 = b_lin1.reshape(1, _D)
    bs1 = b_self1.reshape(1, _D)
    bl2 = b_lin2.reshape(1, _D)
    bs2 = b_self2.reshape(1, _D)
    bm1 = b_mp1.reshape(1, _D)
    bm2 = b_mp2.reshape(1, _D)

    m, s = _lin_pair_first(x, W_lin0, bl0, W_self0, bs0)
    agg = _segsum(m, src_p, dst_p)
    m, s = _lin_pair_next(s, agg, W_lin1, bl1, W_self1, bs1)
    agg = _segsum(m, src_p, dst_p)
    m, s = _lin_pair_next(s, agg, W_lin2, bl2, W_self2, bs2)
    agg = _segsum(m, src_p, dst_p)
    emb, out = _head(s, agg, W_mp1, bm1, W_mp2, bm2)
    return (emb, out)
